# baseline (speedup 1.0000x reference)
import functools
import numpy as np
import jax
import jax.numpy as jnp
from jax import lax
from jax.experimental import pallas as pl
from jax.experimental.pallas import tpu as pltpu
from jax.experimental.pallas import tpu_sc as plsc

_B, _D = 16384, 128
_SCALE = float(_D * 0.3)


def _packed_rank_words() -> np.ndarray:
    rand = jax.random.uniform(jax.random.key(12345), (_B, _D), jnp.float32)
    order = jnp.argsort(rand, axis=1)
    ranks = np.asarray(jnp.argsort(order, axis=1)).astype(np.uint32)
    r4 = ranks.reshape(_B, 2, 4, 16)
    words = r4[:, :, 0] | (r4[:, :, 1] << 8) | (r4[:, :, 2] << 16) | (r4[:, :, 3] << 24)
    return words.reshape(_B * 32).astype(np.uint32).view(np.int32)


_WORDS_I32 = _packed_rank_words()

_NC = 2
_NS = 16
_NW = _NC * _NS
_RPW = _B // _NW


def _sc_body(emb_hbm, df_hbm, tok_hbm, words_hbm, out_hbm, emb_v, w_v, df_v, tok_v):
    wid = lax.axis_index("s") * _NC + lax.axis_index("c")
    base = wid * _RPW
    pltpu.sync_copy(emb_hbm.at[pl.ds(base * _D, _RPW * _D)], emb_v)
    pltpu.sync_copy(words_hbm.at[pl.ds(base * 32, _RPW * 32)], w_v)
    pltpu.sync_copy(df_hbm.at[pl.ds(base, _RPW)], df_v)
    pltpu.sync_copy(tok_hbm, tok_v)
    toks = [tok_v[pl.ds(16 * j, 16)] for j in range(8)]
    byte = jnp.full((16,), 255, jnp.int32)

    ones = jnp.full((16,), 1, jnp.int32)

    def group(gi, carry):
        fvec = df_v[pl.ds(16 * gi, 16)]
        nmvec = jnp.maximum((fvec * jnp.float32(_SCALE)).astype(jnp.int32), ones)
        for l in range(16):
            nm = jnp.full((16,), nmvec[l], jnp.int32)
            for g in range(2):
                w = w_v[pl.ds(gi * 512 + l * 32 + 16 * g, 16)]
                for k in range(4):
                    j = 4 * g + k
                    rk = lax.shift_right_logical(w, jnp.full((16,), 8 * k, jnp.int32)) & byte
                    off = gi * 2048 + l * _D + 16 * j
                    emb_v[pl.ds(off, 16)] = jnp.where(
                        rk >= nm, emb_v[pl.ds(off, 16)], toks[j]
                    )
        return carry

    lax.fori_loop(0, _RPW // 16, group, 0)
    pltpu.sync_copy(emb_v, out_hbm.at[pl.ds(base * _D, _RPW * _D)])


@jax.jit
def _masked_embeds(emb_flat, df, tok_flat, words):
    mesh = plsc.VectorSubcoreMesh(core_axis_name="c", subcore_axis_name="s")
    call = functools.partial(
        pl.kernel,
        out_type=jax.ShapeDtypeStruct((_B * _D,), jnp.float32),
        mesh=mesh,
        scratch_types=[
            pltpu.VMEM((_RPW * _D,), jnp.float32),
            pltpu.VMEM((_RPW * 32,), jnp.int32),
            pltpu.VMEM((_RPW,), jnp.float32),
            pltpu.VMEM((_D,), jnp.float32),
        ],
    )(_sc_body)
    return call(emb_flat, df, tok_flat, words)


def kernel(embeds, dynamic_factors, mask_token):
    words = jnp.asarray(_WORDS_I32)
    out = _masked_embeds(
        embeds.reshape(_B * _D), dynamic_factors, mask_token.reshape(_D), words
    )
    return out.reshape(_B, _D)


# speedup vs baseline: 2.1492x; 2.1492x over previous
import numpy as np
import jax
import jax.numpy as jnp
from jax.experimental import pallas as pl

_B, _D = 16384, 128
_SCALE = float(_D * 0.3)

def _compute_ranks_i8() -> np.ndarray:
    rand = jax.random.uniform(jax.random.key(12345), (_B, _D), jnp.float32)
    order = jnp.argsort(rand, axis=1)
    ranks = jnp.argsort(order, axis=1)
    return np.asarray(ranks).astype(np.int8)


_RANKS_I8 = _compute_ranks_i8()


_BLK = 4096


def _body(df_ref, emb_ref, ranks_ref, tok_ref, out_ref):
    nm = jnp.maximum(jnp.floor(jnp.float32(_SCALE) * df_ref[...]), 1.0)
    keep = ranks_ref[...].astype(jnp.float32) >= nm
    out_ref[...] = jnp.where(keep, emb_ref[...], tok_ref[...])


def kernel(embeds, dynamic_factors, mask_token):
    ranks = jnp.asarray(_RANKS_I8)
    df2 = dynamic_factors.reshape(_B, 1)
    return pl.pallas_call(
        _body,
        grid=(_B // _BLK,),
        in_specs=[
            pl.BlockSpec((_BLK, 1), lambda i: (i, 0)),
            pl.BlockSpec((_BLK, _D), lambda i: (i, 0)),
            pl.BlockSpec((_BLK, _D), lambda i: (i, 0)),
            pl.BlockSpec((1, _D), lambda i: (0, 0)),
        ],
        out_specs=pl.BlockSpec((_BLK, _D), lambda i: (i, 0)),
        out_shape=jax.ShapeDtypeStruct((_B, _D), jnp.float32),
    )(df2, embeds, ranks, mask_token)
